# baseline (speedup 1.0000x reference)
import functools

import jax
import jax.numpy as jnp
from jax import lax
from jax.experimental import pallas as pl
from jax.experimental.pallas import tpu as pltpu
from jax.experimental.pallas import tpu_sc as plsc

_NC = 2
_NS = 16
_CHUNK = 16
_NBUF = 6
_LAG = 2
_SP_CHUNK = 56
_SP_ISS = 4
_SP_DEPTH = 2
_SP_PER_TILE = 8
_SP_ROWS = _SP_CHUNK * _SP_ISS * _SP_PER_TILE


def _sc_copy_body(w_hbm, out_hbm, buf, spbuf, isem, osem, sp_isem, sp_osem, *,
                  rows_per_c):
    cid = lax.axis_index("c")
    sid = lax.axis_index("s")
    stream_rows = rows_per_c - _SP_ROWS
    rows_per_t = stream_rows // _NS
    base = cid * rows_per_c + sid * rows_per_t
    n = rows_per_t // _CHUNK

    def chunk(i):
        return pl.ds(base + i * _CHUNK, _CHUNK)

    sp_base = cid * rows_per_c + stream_rows
    is_sp = sid < _SP_ISS
    sp_sid = jnp.minimum(sid, _SP_ISS - 1)

    def sp_slice(k):
        ci = k * _SP_ISS + sp_sid
        return pl.ds(sp_base + ci * _SP_CHUNK, _SP_CHUNK)

    def sp_slot(k):
        return sp_sid * _SP_DEPTH + (k % _SP_DEPTH)

    sp_in = [
        pltpu.make_async_copy(w_hbm.at[sp_slice(k)], spbuf.at[sp_slot(k)],
                              sp_isem.at[k % _SP_DEPTH])
        for k in range(_SP_PER_TILE)
    ]
    sp_out = [
        pltpu.make_async_copy(spbuf.at[sp_slot(k)], out_hbm.at[0, sp_slice(k)],
                              sp_osem.at[k % _SP_DEPTH])
        for k in range(_SP_PER_TILE)
    ]

    def sp_step(k):
        @pl.when(is_sp)
        def _():
            sp_in[k].wait()
            sp_out[k].start()
            j = k + _SP_DEPTH
            if j < _SP_PER_TILE:
                sp_out[k].wait()
                sp_in[j].start()

    @pl.when(is_sp)
    def _():
        for k in range(_SP_DEPTH):
            sp_in[k].start()

    in_cp = [
        pltpu.make_async_copy(w_hbm.at[chunk(i)], buf.at[i % _NBUF], isem.at[i % _NBUF])
        for i in range(n)
    ]
    out_cp = [
        pltpu.make_async_copy(buf.at[i % _NBUF], out_hbm.at[0, chunk(i)], osem.at[i % _NBUF])
        for i in range(n)
    ]

    for i in range(min(_NBUF, n)):
        in_cp[i].start()
    for i in range(n):
        in_cp[i].wait()
        out_cp[i].start()
        if i < _SP_PER_TILE:
            sp_step(i)
        if i >= _LAG:
            out_cp[i - _LAG].wait()
            j = i - _LAG + _NBUF
            if j < n:
                in_cp[j].start()
    for i in range(max(0, n - _LAG), n):
        out_cp[i].wait()
    for i in range(n, _SP_PER_TILE):
        sp_step(i)

    @pl.when(is_sp)
    def _():
        for k in range(max(0, _SP_PER_TILE - _SP_DEPTH), _SP_PER_TILE):
            sp_out[k].wait()


def kernel(x, W):
    del x
    rows, dim = W.shape
    rows_per_c = rows // _NC
    mesh = plsc.VectorSubcoreMesh(core_axis_name="c", subcore_axis_name="s")
    sc_copy = functools.partial(
        pl.kernel,
        mesh=mesh,
        out_type=jax.ShapeDtypeStruct((1, rows, dim), W.dtype),
        scratch_types=[
            pltpu.VMEM((_NBUF, _CHUNK, dim), W.dtype),
            pltpu.VMEM_SHARED((_SP_ISS * _SP_DEPTH, _SP_CHUNK, dim), W.dtype),
            pltpu.SemaphoreType.DMA((_NBUF,)),
            pltpu.SemaphoreType.DMA((_NBUF,)),
            pltpu.SemaphoreType.DMA((_SP_DEPTH,)),
            pltpu.SemaphoreType.DMA((_SP_DEPTH,)),
        ],
    )(functools.partial(_sc_copy_body, rows_per_c=rows_per_c))
    return sc_copy(W)


# speedup vs baseline: 1.0273x; 1.0273x over previous
import functools

import jax
import jax.numpy as jnp
from jax import lax
from jax.experimental import pallas as pl
from jax.experimental.pallas import tpu as pltpu
from jax.experimental.pallas import tpu_sc as plsc

_NC = 2
_NS = 16
_NW = _NC * _NS
_CHUNK = 16
_NBUF = 6
_LAG = 2


def _sc_copy_body(w_hbm, out_hbm, buf, isem, osem, *, rows_per_w):
    wid = lax.axis_index("s") * _NC + lax.axis_index("c")
    base = wid * rows_per_w
    n = rows_per_w // _CHUNK

    def chunk(i):
        return pl.ds(base + i * _CHUNK, _CHUNK)

    in_cp = [
        pltpu.make_async_copy(w_hbm.at[chunk(i)], buf.at[i % _NBUF], isem.at[i % _NBUF])
        for i in range(n)
    ]
    out_cp = [
        pltpu.make_async_copy(buf.at[i % _NBUF], out_hbm.at[0, chunk(i)], osem.at[i % _NBUF])
        for i in range(n)
    ]

    for i in range(min(_NBUF, n)):
        in_cp[i].start()
    for i in range(n):
        in_cp[i].wait()
        out_cp[i].start()
        if i >= _LAG:
            out_cp[i - _LAG].wait()
            j = i - _LAG + _NBUF
            if j < n:
                in_cp[j].start()
    for i in range(max(0, n - _LAG), n):
        out_cp[i].wait()


def kernel(x, W):
    del x
    rows, dim = W.shape
    rows_per_w = rows // _NW
    mesh = plsc.VectorSubcoreMesh(core_axis_name="c", subcore_axis_name="s")
    sc_copy = functools.partial(
        pl.kernel,
        mesh=mesh,
        out_type=jax.ShapeDtypeStruct((1, rows, dim), W.dtype),
        scratch_types=[
            pltpu.VMEM((_NBUF, _CHUNK, dim), W.dtype),
            pltpu.SemaphoreType.DMA((_NBUF,)),
            pltpu.SemaphoreType.DMA((_NBUF,)),
        ],
    )(functools.partial(_sc_copy_body, rows_per_w=rows_per_w))
    return sc_copy(W)
